# Initial kernel scaffold; baseline (speedup 1.0000x reference)
import jax
import jax.numpy as jnp
from jax.experimental import pallas as pl


def kernel(x, edge_index, edge_type, w1, root1, b1, w2, root2, b2, lw, lb):
    raise NotImplementedError("write your pallas kernel here")


# speedup vs baseline: 5.0300x; 5.0300x over previous
import jax
import jax.numpy as jnp
from jax import lax
from jax.experimental import pallas as pl
from jax.experimental.pallas import tpu as pltpu
from jax.experimental.pallas import tpu_sc as plsc

_N = 10000
_E = 320000
_R = 8
_D = 128
_DH = 32
_NH = _D // _DH
_LOUT = 40
_NC = 2
_NS = 16
_NW = _NC * _NS
_EPT = _E // _NW
_K = 80
_NCH = _EPT // _K
_NR = _N * _R
_ZR = 1600
_NZC = _NR // _ZR
_WB = 200
_NWC = _N // _WB
_BN = 1000

_f32 = jnp.float32
_i32 = jnp.int32

_mesh = plsc.VectorSubcoreMesh(
    core_axis_name="c", subcore_axis_name="s", num_cores=_NC, num_subcores=_NS
)
_sc_params = pltpu.CompilerParams(needs_layout_passes=False)
_sc_params_linear = pltpu.CompilerParams(
    needs_layout_passes=False, use_tc_tiling_on_sc=False)


def _counts_body(sidx_hbm, cnt_out, ones_v, sidx_v, zbuf, cnt_sp):
    c = lax.axis_index("c")
    s = lax.axis_index("s")
    wid = s * _NC + c

    def zrow(i, carry):
        zbuf[pl.ds(i * 16, 16)] = jnp.zeros((16,), _f32)
        return carry

    lax.fori_loop(0, _ZR // 16, zrow, 0)

    def zcnt(i, carry):
        cid = i * _NS + s

        @pl.when(cid < _NZC)
        def _():
            pltpu.sync_copy(zbuf, cnt_sp.at[pl.ds(cid * _ZR, _ZR)])

        return carry

    lax.fori_loop(0, (_NZC + _NS - 1) // _NS, zcnt, 0)

    def orow(i, carry):
        ones_v[pl.ds(i * 16, 16)] = jnp.ones((16,), _f32)
        return carry

    lax.fori_loop(0, _K // 16, orow, 0)

    plsc.subcore_barrier()

    base = wid * _EPT

    def chunk(ci, carry):
        pltpu.sync_copy(sidx_hbm.at[pl.ds(base + ci * _K, _K)], sidx_v)
        pltpu.sync_copy(ones_v, cnt_sp.at[sidx_v], add=True)
        return carry

    lax.fori_loop(0, _NCH, chunk, 0)

    plsc.subcore_barrier()

    def wout(i, carry):
        cid = i * _NS + s

        @pl.when(cid < _NZC)
        def _():
            off = cid * _ZR
            pltpu.sync_copy(cnt_sp.at[pl.ds(off, _ZR)], zbuf)
            pltpu.sync_copy(zbuf, cnt_out.at[pl.ds(c * _NR + off, _ZR)])

        return carry

    lax.fori_loop(0, (_NZC + _NS - 1) // _NS, wout, 0)


def _counts(sidx):
    f = pl.kernel(
        _counts_body,
        out_type=jax.ShapeDtypeStruct((_NC * _NR,), _f32),
        mesh=_mesh,
        compiler_params=_sc_params,
        scratch_types=[
            pltpu.VMEM((_K,), _f32),
            pltpu.VMEM((_K,), _i32),
            pltpu.VMEM((_ZR,), _f32),
            pltpu.VMEM_SHARED((_NR,), _f32),
        ],
    )
    return f(sidx)


def _msg_body(y0_hbm, y1_hbm, y2_hbm, y3_hbm, row_hbm, sidx_hbm, gidx_hbm,
              inv_hbm, out_hbm, inv_v, ridx_v, sidx_v, gidx_v, scale_v,
              rows_v, wbuf, sem, acc_sp):
    c = lax.axis_index("c")
    s = lax.axis_index("s")
    wid = s * _NC + c
    base = wid * _EPT

    def zrow(i, carry):
        for dd in range(_DH // 16):
            wbuf[i, pl.ds(dd * 16, 16)] = jnp.zeros((16,), _f32)
        return carry

    lax.fori_loop(0, _WB, zrow, 0)

    pltpu.sync_copy(inv_hbm, inv_v)

    for h, y_hbm in enumerate((y0_hbm, y1_hbm, y2_hbm, y3_hbm)):

        def zacc(i, carry):
            cid = i * _NS + s

            @pl.when(cid < _NWC)
            def _():
                pltpu.sync_copy(wbuf, acc_sp.at[pl.ds(cid * _WB, _WB)])

            return carry

        lax.fori_loop(0, (_NWC + _NS - 1) // _NS, zacc, 0)
        plsc.subcore_barrier()

        def chunk(ci, carry):
            off = base + ci * _K
            pltpu.sync_copy(row_hbm.at[pl.ds(off, _K)], ridx_v)
            pltpu.sync_copy(sidx_hbm.at[pl.ds(off, _K)], sidx_v)
            pltpu.sync_copy(gidx_hbm.at[pl.ds(off, _K)], gidx_v)
            pltpu.async_copy(y_hbm.at[gidx_v], rows_v, sem).wait()

            def sgrp(i, cc):
                sv = plsc.load_gather(inv_v, [sidx_v[pl.ds(i * 16, 16)]])
                scale_v[pl.ds(i * 16, 16)] = sv
                return cc

            lax.fori_loop(0, _K // 16, sgrp, 0)

            def emul(j, cc):
                sj = plsc.load_gather(scale_v, [jnp.full((16,), j, _i32)])
                for dd in range(_DH // 16):
                    sl = pl.ds(dd * 16, 16)
                    rows_v[j, sl] = rows_v[j, sl] * sj
                return cc

            lax.fori_loop(0, _K, emul, 0)

            pltpu.sync_copy(rows_v, acc_sp.at[ridx_v], add=True)
            return carry

        lax.fori_loop(0, _NCH, chunk, 0)
        plsc.subcore_barrier()

        def wout(i, carry):
            cid = i * _NS + s

            @pl.when(cid < _NWC)
            def _():
                pltpu.sync_copy(acc_sp.at[pl.ds(cid * _WB, _WB)], wbuf)
                pltpu.sync_copy(
                    wbuf, out_hbm.at[h * _NC + c, pl.ds(cid * _WB, _WB)])

            return carry

        lax.fori_loop(0, (_NWC + _NS - 1) // _NS, wout, 0)
        plsc.subcore_barrier()

        if h < _NH - 1:
            lax.fori_loop(0, _WB, zrow, 0)


def _msg(ys, row, sidx, gidx, inv):
    f = pl.kernel(
        _msg_body,
        out_type=jax.ShapeDtypeStruct((_NH * _NC, _N, _DH), _f32),
        mesh=_mesh,
        compiler_params=_sc_params_linear,
        scratch_types=[
            pltpu.VMEM((_NR,), _f32),
            pltpu.VMEM((_K,), _i32),
            pltpu.VMEM((_K,), _i32),
            pltpu.VMEM((_K,), _i32),
            pltpu.VMEM((_K,), _f32),
            pltpu.VMEM((_K, _DH), _f32),
            pltpu.VMEM((_WB, _DH), _f32),
            pltpu.SemaphoreType.DMA,
            pltpu.VMEM_SHARED((_N, _DH), _f32),
        ],
    )
    return f(*ys, row, sidx, gidx, inv)


def _inv_body(c0_ref, c1_ref, inv_ref):
    csum = c0_ref[...] + c1_ref[...]
    inv_ref[...] = 1.0 / jnp.maximum(csum, 1.0)


def _tc_inv(c0, c1):
    return pl.pallas_call(
        _inv_body,
        out_shape=jax.ShapeDtypeStruct((_NR // _D, _D), _f32),
    )(c0, c1)


def _lin_in_body(x_ref, w_ref, root_ref, b_ref, *out_refs):
    y_refs, ro_ref = out_refs[:_NH], out_refs[_NH]
    h = x_ref[...]
    for r in range(_R):
        yr = jnp.dot(h, w_ref[r], preferred_element_type=_f32)
        for q in range(_NH):
            y_refs[q][r] = yr[:, q * _DH:(q + 1) * _DH]
    ro_ref[...] = jnp.dot(h, root_ref[...], preferred_element_type=_f32) + b_ref[...]


def _lin_mid_body(msg_ref, ro1_ref, w_ref, root_ref, b_ref, *out_refs):
    y_refs, ro_ref = out_refs[:_NH], out_refs[_NH]
    parts = [msg_ref[2 * q] + msg_ref[2 * q + 1] for q in range(_NH)]
    h = jnp.maximum(
        jnp.concatenate(parts, axis=-1) + ro1_ref[...], 0.0)
    for r in range(_R):
        yr = jnp.dot(h, w_ref[r], preferred_element_type=_f32)
        for q in range(_NH):
            y_refs[q][r] = yr[:, q * _DH:(q + 1) * _DH]
    ro_ref[...] = jnp.dot(h, root_ref[...], preferred_element_type=_f32) + b_ref[...]


_LAYER_OUT_SPECS = (
    [pl.BlockSpec((_R, _BN, _DH), lambda i: (0, i, 0)) for _ in range(_NH)]
    + [pl.BlockSpec((_BN, _D), lambda i: (i, 0))])
_LAYER_OUT_SHAPE = (
    [jax.ShapeDtypeStruct((_R, _N, _DH), _f32) for _ in range(_NH)]
    + [jax.ShapeDtypeStruct((_N, _D), _f32)])


def _tc_layer_in(x, w, root, b):
    return pl.pallas_call(
        _lin_in_body,
        grid=(_N // _BN,),
        in_specs=[
            pl.BlockSpec((_BN, _D), lambda i: (i, 0)),
            pl.BlockSpec((_R, _D, _D), lambda i: (0, 0, 0)),
            pl.BlockSpec((_D, _D), lambda i: (0, 0)),
            pl.BlockSpec((1, _D), lambda i: (0, 0)),
        ],
        out_specs=_LAYER_OUT_SPECS,
        out_shape=_LAYER_OUT_SHAPE,
    )(x, w, root, b)


def _tc_layer_mid(msg, ro1, w, root, b):
    return pl.pallas_call(
        _lin_mid_body,
        grid=(_N // _BN,),
        in_specs=[
            pl.BlockSpec((_NH * _NC, _BN, _DH), lambda i: (0, i, 0)),
            pl.BlockSpec((_BN, _D), lambda i: (i, 0)),
            pl.BlockSpec((_R, _D, _D), lambda i: (0, 0, 0)),
            pl.BlockSpec((_D, _D), lambda i: (0, 0)),
            pl.BlockSpec((1, _D), lambda i: (0, 0)),
        ],
        out_specs=_LAYER_OUT_SPECS,
        out_shape=_LAYER_OUT_SHAPE,
    )(msg, ro1, w, root, b)


def _head_body(msg_ref, ro_ref, lw_ref, lb_ref, o_ref):
    parts = [msg_ref[2 * q] + msg_ref[2 * q + 1] for q in range(_NH)]
    h = jnp.maximum(
        jnp.concatenate(parts, axis=-1) + ro_ref[...], 0.0)
    logits = jnp.dot(h, lw_ref[...], preferred_element_type=_f32) + lb_ref[...]
    m = jnp.max(logits, axis=1, keepdims=True)
    e = jnp.exp(logits - m)
    lse = jnp.log(jnp.sum(e, axis=1, keepdims=True)) + m
    o_ref[...] = logits - lse


def _tc_head(msg, ro, lw, lb):
    return pl.pallas_call(
        _head_body,
        grid=(_N // _BN,),
        in_specs=[
            pl.BlockSpec((_NH * _NC, _BN, _DH), lambda i: (0, i, 0)),
            pl.BlockSpec((_BN, _D), lambda i: (i, 0)),
            pl.BlockSpec((_D, _LOUT), lambda i: (0, 0)),
            pl.BlockSpec((1, _LOUT), lambda i: (0, 0)),
        ],
        out_specs=pl.BlockSpec((_BN, _LOUT), lambda i: (i, 0)),
        out_shape=jax.ShapeDtypeStruct((_N, _LOUT), _f32),
    )(msg, ro, lw, lb)


def kernel(x, edge_index, edge_type, w1, root1, b1, w2, root2, b2, lw, lb):
    row = edge_index[0]
    col = edge_index[1]
    et = edge_type
    sidx = row * _R + et
    gidx = et * _N + col

    counts = _counts(sidx).reshape(_NC, _NR // _D, _D)
    inv = _tc_inv(counts[0], counts[1]).reshape(_NR)

    *y1s, ro1 = _tc_layer_in(x, w1, root1, b1.reshape(1, _D))
    msg1 = _msg([y.reshape(_NR, _DH) for y in y1s], row, sidx, gidx, inv)
    *y2s, ro2 = _tc_layer_mid(msg1, ro1, w2, root2, b2.reshape(1, _D))
    msg2 = _msg([y.reshape(_NR, _DH) for y in y2s], row, sidx, gidx, inv)
    return _tc_head(msg2, ro2, lw, lb.reshape(1, _LOUT))


# speedup vs baseline: 13.9547x; 2.7743x over previous
import jax
import jax.numpy as jnp
from jax import lax
from jax.experimental import pallas as pl
from jax.experimental.pallas import tpu as pltpu
from jax.experimental.pallas import tpu_sc as plsc

_N = 10000
_E = 320000
_R = 8
_D = 128
_LOUT = 40
_NC = 2
_NS = 16
_NW = _NC * _NS
_EPT = _E // _NW
_K = 80
_NCH = _EPT // _K
_NR = _N * _R
_ZR = 1600
_NZC = _NR // _ZR
_WB = 100
_NWC = _N // _WB
_BN = 1000

_f32 = jnp.float32
_i32 = jnp.int32

_mesh = plsc.VectorSubcoreMesh(
    core_axis_name="c", subcore_axis_name="s", num_cores=_NC, num_subcores=_NS
)
_sc_params = pltpu.CompilerParams(needs_layout_passes=False)
_sc_params_linear = pltpu.CompilerParams(
    needs_layout_passes=False, use_tc_tiling_on_sc=False)


def _counts_body(sidx_hbm, cnt_out, ones_v, sidx_v, zbuf, cnt_sp):
    c = lax.axis_index("c")
    s = lax.axis_index("s")
    wid = s * _NC + c

    def zrow(i, carry):
        zbuf[pl.ds(i * 16, 16)] = jnp.zeros((16,), _f32)
        return carry

    lax.fori_loop(0, _ZR // 16, zrow, 0)

    def zcnt(i, carry):
        cid = i * _NS + s

        @pl.when(cid < _NZC)
        def _():
            pltpu.sync_copy(zbuf, cnt_sp.at[pl.ds(cid * _ZR, _ZR)])

        return carry

    lax.fori_loop(0, (_NZC + _NS - 1) // _NS, zcnt, 0)

    def orow(i, carry):
        ones_v[pl.ds(i * 16, 16)] = jnp.ones((16,), _f32)
        return carry

    lax.fori_loop(0, _K // 16, orow, 0)

    plsc.subcore_barrier()

    base = wid * _EPT

    def chunk(ci, carry):
        pltpu.sync_copy(sidx_hbm.at[pl.ds(base + ci * _K, _K)], sidx_v)
        pltpu.sync_copy(ones_v, cnt_sp.at[sidx_v], add=True)
        return carry

    lax.fori_loop(0, _NCH, chunk, 0)

    plsc.subcore_barrier()

    def wout(i, carry):
        cid = i * _NS + s

        @pl.when(cid < _NZC)
        def _():
            off = cid * _ZR
            pltpu.sync_copy(cnt_sp.at[pl.ds(off, _ZR)], zbuf)
            pltpu.sync_copy(zbuf, cnt_out.at[pl.ds(c * _NR + off, _ZR)])

        return carry

    lax.fori_loop(0, (_NZC + _NS - 1) // _NS, wout, 0)


def _counts(sidx):
    f = pl.kernel(
        _counts_body,
        out_type=jax.ShapeDtypeStruct((_NC * _NR,), _f32),
        mesh=_mesh,
        compiler_params=_sc_params,
        scratch_types=[
            pltpu.VMEM((_K,), _f32),
            pltpu.VMEM((_K,), _i32),
            pltpu.VMEM((_ZR,), _f32),
            pltpu.VMEM_SHARED((_NR,), _f32),
        ],
    )
    return f(sidx)


def _scales_body(sidx_hbm, inv_hbm, s_out, inv_v, sidx_v, scale_v):
    c = lax.axis_index("c")
    s = lax.axis_index("s")
    wid = s * _NC + c
    base = wid * _EPT

    pltpu.sync_copy(inv_hbm, inv_v)

    def chunk(ci, carry):
        off = base + ci * _K
        pltpu.sync_copy(sidx_hbm.at[pl.ds(off, _K)], sidx_v)

        def sgrp(i, cc):
            sv = plsc.load_gather(inv_v, [sidx_v[pl.ds(i * 16, 16)]])
            scale_v[pl.ds(i * 16, 16)] = sv
            return cc

        lax.fori_loop(0, _K // 16, sgrp, 0)
        pltpu.sync_copy(scale_v, s_out.at[pl.ds(off, _K)])
        return carry

    lax.fori_loop(0, _NCH, chunk, 0)


def _scales(sidx, inv):
    f = pl.kernel(
        _scales_body,
        out_type=jax.ShapeDtypeStruct((_E,), _f32),
        mesh=_mesh,
        compiler_params=_sc_params_linear,
        scratch_types=[
            pltpu.VMEM((_NR,), _f32),
            pltpu.VMEM((_K,), _i32),
            pltpu.VMEM((_K,), _f32),
        ],
    )
    return f(sidx, inv)


def _msg_body(y_hbm, row_hbm, gidx_hbm, s_hbm, out_hbm,
              ridx_v, gidx_v, scale_v, rows_v, wbuf, sem, acc_sp):
    c = lax.axis_index("c")
    s = lax.axis_index("s")
    wid = s * _NC + c
    base = wid * _EPT

    def zrow(i, carry):
        for dd in range(_D // 16):
            wbuf[i, pl.ds(dd * 16, 16)] = jnp.zeros((16,), _f32)
        return carry

    lax.fori_loop(0, _WB, zrow, 0)

    def zacc(i, carry):
        cid = i * _NS + s

        @pl.when(cid < _NWC)
        def _():
            pltpu.sync_copy(wbuf, acc_sp.at[pl.ds(cid * _WB, _WB)])

        return carry

    lax.fori_loop(0, (_NWC + _NS - 1) // _NS, zacc, 0)
    plsc.subcore_barrier()

    def chunk(ci, carry):
        off = base + ci * _K
        pltpu.sync_copy(row_hbm.at[pl.ds(off, _K)], ridx_v)
        pltpu.sync_copy(gidx_hbm.at[pl.ds(off, _K)], gidx_v)
        pltpu.sync_copy(s_hbm.at[pl.ds(off, _K)], scale_v)
        pltpu.async_copy(y_hbm.at[gidx_v], rows_v, sem).wait()

        def emul(j, cc):
            sj = plsc.load_gather(scale_v, [jnp.full((16,), j, _i32)])
            for dd in range(_D // 16):
                sl = pl.ds(dd * 16, 16)
                rows_v[j, sl] = rows_v[j, sl] * sj
            return cc

        lax.fori_loop(0, _K, emul, 0)

        pltpu.sync_copy(rows_v, acc_sp.at[ridx_v], add=True)
        return carry

    lax.fori_loop(0, _NCH, chunk, 0)
    plsc.subcore_barrier()

    def wout(i, carry):
        cid = i * _NS + s

        @pl.when(cid < _NWC)
        def _():
            pltpu.sync_copy(acc_sp.at[pl.ds(cid * _WB, _WB)], wbuf)
            pltpu.sync_copy(wbuf, out_hbm.at[c, pl.ds(cid * _WB, _WB)])

        return carry

    lax.fori_loop(0, (_NWC + _NS - 1) // _NS, wout, 0)


def _msg(y, row, gidx, sedge):
    f = pl.kernel(
        _msg_body,
        out_type=jax.ShapeDtypeStruct((_NC, _N, _D), _f32),
        mesh=_mesh,
        compiler_params=_sc_params_linear,
        scratch_types=[
            pltpu.VMEM((_K,), _i32),
            pltpu.VMEM((_K,), _i32),
            pltpu.VMEM((_K,), _f32),
            pltpu.VMEM((_K, _D), _f32),
            pltpu.VMEM((_WB, _D), _f32),
            pltpu.SemaphoreType.DMA,
            pltpu.VMEM_SHARED((_N, _D), _f32),
        ],
    )
    return f(y, row, gidx, sedge)


def _inv_body(c0_ref, c1_ref, inv_ref):
    csum = c0_ref[...] + c1_ref[...]
    inv_ref[...] = 1.0 / jnp.maximum(csum, 1.0)


def _tc_inv(c0, c1):
    return pl.pallas_call(
        _inv_body,
        out_shape=jax.ShapeDtypeStruct((_NR // _D, _D), _f32),
    )(c0, c1)


def _lin_in_body(x_ref, w_ref, root_ref, b_ref, y_ref, ro_ref):
    h = x_ref[...]
    for r in range(_R):
        y_ref[r] = jnp.dot(h, w_ref[r], preferred_element_type=_f32)
    ro_ref[...] = jnp.dot(h, root_ref[...], preferred_element_type=_f32) + b_ref[...]


def _lin_mid_body(msg_ref, ro1_ref, w_ref, root_ref, b_ref, y_ref, ro_ref):
    h = jnp.maximum(msg_ref[0] + msg_ref[1] + ro1_ref[...], 0.0)
    for r in range(_R):
        y_ref[r] = jnp.dot(h, w_ref[r], preferred_element_type=_f32)
    ro_ref[...] = jnp.dot(h, root_ref[...], preferred_element_type=_f32) + b_ref[...]


_LAYER_OUT_SPECS = [
    pl.BlockSpec((_R, _BN, _D), lambda i: (0, i, 0)),
    pl.BlockSpec((_BN, _D), lambda i: (i, 0)),
]
_LAYER_OUT_SHAPE = [
    jax.ShapeDtypeStruct((_R, _N, _D), _f32),
    jax.ShapeDtypeStruct((_N, _D), _f32),
]


def _tc_layer_in(x, w, root, b):
    return pl.pallas_call(
        _lin_in_body,
        grid=(_N // _BN,),
        in_specs=[
            pl.BlockSpec((_BN, _D), lambda i: (i, 0)),
            pl.BlockSpec((_R, _D, _D), lambda i: (0, 0, 0)),
            pl.BlockSpec((_D, _D), lambda i: (0, 0)),
            pl.BlockSpec((1, _D), lambda i: (0, 0)),
        ],
        out_specs=_LAYER_OUT_SPECS,
        out_shape=_LAYER_OUT_SHAPE,
    )(x, w, root, b)


def _tc_layer_mid(msg, ro1, w, root, b):
    return pl.pallas_call(
        _lin_mid_body,
        grid=(_N // _BN,),
        in_specs=[
            pl.BlockSpec((_NC, _BN, _D), lambda i: (0, i, 0)),
            pl.BlockSpec((_BN, _D), lambda i: (i, 0)),
            pl.BlockSpec((_R, _D, _D), lambda i: (0, 0, 0)),
            pl.BlockSpec((_D, _D), lambda i: (0, 0)),
            pl.BlockSpec((1, _D), lambda i: (0, 0)),
        ],
        out_specs=_LAYER_OUT_SPECS,
        out_shape=_LAYER_OUT_SHAPE,
    )(msg, ro1, w, root, b)


def _head_body(msg_ref, ro_ref, lw_ref, lb_ref, o_ref):
    h = jnp.maximum(msg_ref[0] + msg_ref[1] + ro_ref[...], 0.0)
    logits = jnp.dot(h, lw_ref[...], preferred_element_type=_f32) + lb_ref[...]
    m = jnp.max(logits, axis=1, keepdims=True)
    e = jnp.exp(logits - m)
    lse = jnp.log(jnp.sum(e, axis=1, keepdims=True)) + m
    o_ref[...] = logits - lse


def _tc_head(msg, ro, lw, lb):
    return pl.pallas_call(
        _head_body,
        grid=(_N // _BN,),
        in_specs=[
            pl.BlockSpec((_NC, _BN, _D), lambda i: (0, i, 0)),
            pl.BlockSpec((_BN, _D), lambda i: (i, 0)),
            pl.BlockSpec((_D, _LOUT), lambda i: (0, 0)),
            pl.BlockSpec((1, _LOUT), lambda i: (0, 0)),
        ],
        out_specs=pl.BlockSpec((_BN, _LOUT), lambda i: (i, 0)),
        out_shape=jax.ShapeDtypeStruct((_N, _LOUT), _f32),
    )(msg, ro, lw, lb)


def kernel(x, edge_index, edge_type, w1, root1, b1, w2, root2, b2, lw, lb):
    row = edge_index[0]
    col = edge_index[1]
    et = edge_type
    sidx = row * _R + et
    gidx = et * _N + col

    counts = _counts(sidx).reshape(_NC, _NR // _D, _D)
    inv = _tc_inv(counts[0], counts[1]).reshape(_NR)
    sedge = _scales(sidx, inv)

    y1, ro1 = _tc_layer_in(x, w1, root1, b1.reshape(1, _D))
    msg1 = _msg(y1.reshape(_NR, _D), row, gidx, sedge)
    y2, ro2 = _tc_layer_mid(msg1, ro1, w2, root2, b2.reshape(1, _D))
    msg2 = _msg(y2.reshape(_NR, _D), row, gidx, sedge)
    return _tc_head(msg2, ro2, lw, lb.reshape(1, _LOUT))


# speedup vs baseline: 18.0046x; 1.2902x over previous
import jax
import jax.numpy as jnp
from jax import lax
from jax.experimental import pallas as pl
from jax.experimental.pallas import tpu as pltpu
from jax.experimental.pallas import tpu_sc as plsc

_N = 10000
_E = 320000
_R = 8
_D = 128
_LOUT = 40
_NC = 2
_NS = 16
_NW = _NC * _NS
_EPT = _E // _NW
_K = 80
_NCH = _EPT // _K
_NR = _N * _R
_ZR = 1600
_NZC = _NR // _ZR
_WB = 100
_NWC = _N // _WB
_BN = 1000

_f32 = jnp.float32
_i32 = jnp.int32

_mesh = plsc.VectorSubcoreMesh(
    core_axis_name="c", subcore_axis_name="s", num_cores=_NC, num_subcores=_NS
)
_sc_params = pltpu.CompilerParams(needs_layout_passes=False)
_sc_params_linear = pltpu.CompilerParams(
    needs_layout_passes=False, use_tc_tiling_on_sc=False)


def _counts_body(sidx_hbm, cnt_out, ones_v, sidx_v, zbuf, cnt_sp):
    c = lax.axis_index("c")
    s = lax.axis_index("s")
    wid = s * _NC + c

    def zrow(i, carry):
        zbuf[pl.ds(i * 16, 16)] = jnp.zeros((16,), _f32)
        return carry

    lax.fori_loop(0, _ZR // 16, zrow, 0)

    def zcnt(i, carry):
        cid = i * _NS + s

        @pl.when(cid < _NZC)
        def _():
            pltpu.sync_copy(zbuf, cnt_sp.at[pl.ds(cid * _ZR, _ZR)])

        return carry

    lax.fori_loop(0, (_NZC + _NS - 1) // _NS, zcnt, 0)

    def orow(i, carry):
        ones_v[pl.ds(i * 16, 16)] = jnp.ones((16,), _f32)
        return carry

    lax.fori_loop(0, _K // 16, orow, 0)

    plsc.subcore_barrier()

    base = wid * _EPT

    def chunk(ci, carry):
        pltpu.sync_copy(sidx_hbm.at[pl.ds(base + ci * _K, _K)], sidx_v)
        pltpu.sync_copy(ones_v, cnt_sp.at[sidx_v], add=True)
        return carry

    lax.fori_loop(0, _NCH, chunk, 0)

    plsc.subcore_barrier()

    def wout(i, carry):
        cid = i * _NS + s

        @pl.when(cid < _NZC)
        def _():
            off = cid * _ZR
            pltpu.sync_copy(cnt_sp.at[pl.ds(off, _ZR)], zbuf)
            pltpu.sync_copy(zbuf, cnt_out.at[pl.ds(c * _NR + off, _ZR)])

        return carry

    lax.fori_loop(0, (_NZC + _NS - 1) // _NS, wout, 0)


def _counts(sidx):
    f = pl.kernel(
        _counts_body,
        out_type=jax.ShapeDtypeStruct((_NC * _NR,), _f32),
        mesh=_mesh,
        compiler_params=_sc_params,
        scratch_types=[
            pltpu.VMEM((_K,), _f32),
            pltpu.VMEM((_K,), _i32),
            pltpu.VMEM((_ZR,), _f32),
            pltpu.VMEM_SHARED((_NR,), _f32),
        ],
    )
    return f(sidx)


def _scales_body(sidx_hbm, inv_hbm, s_out, inv_v, sidx_v, scale_v):
    c = lax.axis_index("c")
    s = lax.axis_index("s")
    wid = s * _NC + c
    base = wid * _EPT

    pltpu.sync_copy(inv_hbm, inv_v)

    def chunk(ci, carry):
        off = base + ci * _K
        pltpu.sync_copy(sidx_hbm.at[pl.ds(off, _K)], sidx_v)

        def sgrp(i, cc):
            sv = plsc.load_gather(inv_v, [sidx_v[pl.ds(i * 16, 16)]])
            scale_v[pl.ds(i * 16, 16)] = sv
            return cc

        lax.fori_loop(0, _K // 16, sgrp, 0)
        pltpu.sync_copy(scale_v, s_out.at[pl.ds(off, _K)])
        return carry

    lax.fori_loop(0, _NCH, chunk, 0)


def _scales(sidx, inv):
    f = pl.kernel(
        _scales_body,
        out_type=jax.ShapeDtypeStruct((_E,), _f32),
        mesh=_mesh,
        compiler_params=_sc_params_linear,
        scratch_types=[
            pltpu.VMEM((_NR,), _f32),
            pltpu.VMEM((_K,), _i32),
            pltpu.VMEM((_K,), _f32),
        ],
    )
    return f(sidx, inv)


def _msg_body(y_hbm, row_hbm, gidx_hbm, s_hbm, out_hbm,
              ridx_v0, ridx_v1, gidx_v0, gidx_v1, scale_v0, scale_v1,
              rows_v0, rows_v1, wbuf, sem0, sem1, acc_sp):
    c = lax.axis_index("c")
    s = lax.axis_index("s")
    wid = s * _NC + c
    base = wid * _EPT

    ridx = (ridx_v0, ridx_v1)
    gidx = (gidx_v0, gidx_v1)
    scale = (scale_v0, scale_v1)
    rows = (rows_v0, rows_v1)
    sem = (sem0, sem1)

    def zrow(i, carry):
        for dd in range(_D // 16):
            wbuf[i, pl.ds(dd * 16, 16)] = jnp.zeros((16,), _f32)
        return carry

    lax.fori_loop(0, _WB, zrow, 0)

    def zacc(i, carry):
        cid = i * _NS + s

        @pl.when(cid < _NWC)
        def _():
            pltpu.sync_copy(wbuf, acc_sp.at[pl.ds(cid * _WB, _WB)])

        return carry

    lax.fori_loop(0, (_NWC + _NS - 1) // _NS, zacc, 0)
    plsc.subcore_barrier()

    def load_and_start(ci, b):
        off = base + ci * _K
        pltpu.sync_copy(row_hbm.at[pl.ds(off, _K)], ridx[b])
        pltpu.sync_copy(gidx_hbm.at[pl.ds(off, _K)], gidx[b])
        pltpu.sync_copy(s_hbm.at[pl.ds(off, _K)], scale[b])
        pltpu.async_copy(y_hbm.at[gidx[b]], rows[b], sem[b])

    def finish(b):
        pltpu.make_async_copy(y_hbm.at[gidx[b]], rows[b], sem[b]).wait()

        def emul(j, cc):
            sj = plsc.load_gather(scale[b], [jnp.full((16,), j, _i32)])
            for dd in range(_D // 16):
                sl = pl.ds(dd * 16, 16)
                rows[b][j, sl] = rows[b][j, sl] * sj
            return cc

        lax.fori_loop(0, _K, emul, 0)
        pltpu.sync_copy(rows[b], acc_sp.at[ridx[b]], add=True)

    load_and_start(jnp.int32(0), 0)
    load_and_start(jnp.int32(1), 1)

    def outer(i, carry):
        ci0 = i * 2
        for b in range(2):
            ci = ci0 + b
            finish(b)

            @pl.when(ci + 2 < _NCH)
            def _():
                load_and_start(ci + 2, b)

        return carry

    lax.fori_loop(0, (_NCH - 1) // 2, outer, 0)
    finish((_NCH - 1) % 2)
    plsc.subcore_barrier()

    def wout(i, carry):
        cid = i * _NS + s

        @pl.when(cid < _NWC)
        def _():
            pltpu.sync_copy(acc_sp.at[pl.ds(cid * _WB, _WB)], wbuf)
            pltpu.sync_copy(wbuf, out_hbm.at[c, pl.ds(cid * _WB, _WB)])

        return carry

    lax.fori_loop(0, (_NWC + _NS - 1) // _NS, wout, 0)


def _msg(y, row, gidx, sedge):
    f = pl.kernel(
        _msg_body,
        out_type=jax.ShapeDtypeStruct((_NC, _N, _D), _f32),
        mesh=_mesh,
        compiler_params=_sc_params_linear,
        scratch_types=[
            pltpu.VMEM((_K,), _i32),
            pltpu.VMEM((_K,), _i32),
            pltpu.VMEM((_K,), _i32),
            pltpu.VMEM((_K,), _i32),
            pltpu.VMEM((_K,), _f32),
            pltpu.VMEM((_K,), _f32),
            pltpu.VMEM((_K, _D), _f32),
            pltpu.VMEM((_K, _D), _f32),
            pltpu.VMEM((_WB, _D), _f32),
            pltpu.SemaphoreType.DMA,
            pltpu.SemaphoreType.DMA,
            pltpu.VMEM_SHARED((_N, _D), _f32),
        ],
    )
    return f(y, row, gidx, sedge)


def _inv_body(c0_ref, c1_ref, inv_ref):
    csum = c0_ref[...] + c1_ref[...]
    inv_ref[...] = 1.0 / jnp.maximum(csum, 1.0)


def _tc_inv(c0, c1):
    return pl.pallas_call(
        _inv_body,
        out_shape=jax.ShapeDtypeStruct((_NR // _D, _D), _f32),
    )(c0, c1)


def _lin_in_body(x_ref, w_ref, root_ref, b_ref, y_ref, ro_ref):
    h = x_ref[...]
    for r in range(_R):
        y_ref[r] = jnp.dot(h, w_ref[r], preferred_element_type=_f32)
    ro_ref[...] = jnp.dot(h, root_ref[...], preferred_element_type=_f32) + b_ref[...]


def _lin_mid_body(msg_ref, ro1_ref, w_ref, root_ref, b_ref, y_ref, ro_ref):
    h = jnp.maximum(msg_ref[0] + msg_ref[1] + ro1_ref[...], 0.0)
    for r in range(_R):
        y_ref[r] = jnp.dot(h, w_ref[r], preferred_element_type=_f32)
    ro_ref[...] = jnp.dot(h, root_ref[...], preferred_element_type=_f32) + b_ref[...]


_LAYER_OUT_SPECS = [
    pl.BlockSpec((_R, _BN, _D), lambda i: (0, i, 0)),
    pl.BlockSpec((_BN, _D), lambda i: (i, 0)),
]
_LAYER_OUT_SHAPE = [
    jax.ShapeDtypeStruct((_R, _N, _D), _f32),
    jax.ShapeDtypeStruct((_N, _D), _f32),
]


def _tc_layer_in(x, w, root, b):
    return pl.pallas_call(
        _lin_in_body,
        grid=(_N // _BN,),
        in_specs=[
            pl.BlockSpec((_BN, _D), lambda i: (i, 0)),
            pl.BlockSpec((_R, _D, _D), lambda i: (0, 0, 0)),
            pl.BlockSpec((_D, _D), lambda i: (0, 0)),
            pl.BlockSpec((1, _D), lambda i: (0, 0)),
        ],
        out_specs=_LAYER_OUT_SPECS,
        out_shape=_LAYER_OUT_SHAPE,
    )(x, w, root, b)


def _tc_layer_mid(msg, ro1, w, root, b):
    return pl.pallas_call(
        _lin_mid_body,
        grid=(_N // _BN,),
        in_specs=[
            pl.BlockSpec((_NC, _BN, _D), lambda i: (0, i, 0)),
            pl.BlockSpec((_BN, _D), lambda i: (i, 0)),
            pl.BlockSpec((_R, _D, _D), lambda i: (0, 0, 0)),
            pl.BlockSpec((_D, _D), lambda i: (0, 0)),
            pl.BlockSpec((1, _D), lambda i: (0, 0)),
        ],
        out_specs=_LAYER_OUT_SPECS,
        out_shape=_LAYER_OUT_SHAPE,
    )(msg, ro1, w, root, b)


def _head_body(msg_ref, ro_ref, lw_ref, lb_ref, o_ref):
    h = jnp.maximum(msg_ref[0] + msg_ref[1] + ro_ref[...], 0.0)
    logits = jnp.dot(h, lw_ref[...], preferred_element_type=_f32) + lb_ref[...]
    m = jnp.max(logits, axis=1, keepdims=True)
    e = jnp.exp(logits - m)
    lse = jnp.log(jnp.sum(e, axis=1, keepdims=True)) + m
    o_ref[...] = logits - lse


def _tc_head(msg, ro, lw, lb):
    return pl.pallas_call(
        _head_body,
        grid=(_N // _BN,),
        in_specs=[
            pl.BlockSpec((_NC, _BN, _D), lambda i: (0, i, 0)),
            pl.BlockSpec((_BN, _D), lambda i: (i, 0)),
            pl.BlockSpec((_D, _LOUT), lambda i: (0, 0)),
            pl.BlockSpec((1, _LOUT), lambda i: (0, 0)),
        ],
        out_specs=pl.BlockSpec((_BN, _LOUT), lambda i: (i, 0)),
        out_shape=jax.ShapeDtypeStruct((_N, _LOUT), _f32),
    )(msg, ro, lw, lb)


def kernel(x, edge_index, edge_type, w1, root1, b1, w2, root2, b2, lw, lb):
    row = edge_index[0]
    col = edge_index[1]
    et = edge_type
    sidx = row * _R + et
    gidx = et * _N + col

    counts = _counts(sidx).reshape(_NC, _NR // _D, _D)
    inv = _tc_inv(counts[0], counts[1]).reshape(_NR)
    sedge = _scales(sidx, inv)

    y1, ro1 = _tc_layer_in(x, w1, root1, b1.reshape(1, _D))
    msg1 = _msg(y1.reshape(_NR, _D), row, gidx, sedge)
    y2, ro2 = _tc_layer_mid(msg1, ro1, w2, root2, b2.reshape(1, _D))
    msg2 = _msg(y2.reshape(_NR, _D), row, gidx, sedge)
    return _tc_head(msg2, ro2, lw, lb.reshape(1, _LOUT))
